# baseline (speedup 1.0000x reference)
import functools

import jax
import jax.numpy as jnp
from jax import lax
from jax.experimental import pallas as pl
from jax.experimental.pallas import tpu as pltpu
from jax.experimental.pallas import tpu_sc as plsc

N_NODES = 10000
N_EDGES = 320000
D_IN = 128
D_HID = 64
D2 = 16

NC = 2
NS = 16
NW = NC * NS
CH = 128
NP = 10240
NCHT = N_EDGES // CH
FULL = 80
LASTW = NW - 1
LAST = NCHT - LASTW * FULL
RPT = NP // NS

_mesh = plsc.VectorSubcoreMesh(core_axis_name="c", subcore_axis_name="s")
_sc_params = pltpu.CompilerParams(use_tc_tiling_on_sc=False)


def _worker(c, s):
    return s * NC + c


def _load_chunks(ei_hbm, row, w, idx_v):
    @pl.when(w < LASTW)
    def _():
        pltpu.sync_copy(ei_hbm.at[row, pl.ds(w * FULL, FULL)], idx_v)

    @pl.when(w == LASTW)
    def _():
        pltpu.sync_copy(ei_hbm.at[row, pl.ds(LASTW * FULL, LAST)],
                        idx_v.at[pl.ds(0, LAST)])


def _fill(buf, value, d):
    vec = jnp.full((16,), value, jnp.float32)
    nsub = d // 16

    def body(i, carry):
        r = i // nsub
        k = (i % nsub) * 16
        buf[r, pl.ds(k, 16)] = vec
        return carry

    lax.fori_loop(0, CH * nsub, body, 0)


@functools.partial(
    pl.kernel,
    out_type=jax.ShapeDtypeStruct((NC * NP, D2), jnp.float32),
    mesh=_mesh,
    scratch_types=[
        pltpu.VMEM((FULL, CH), jnp.int32),
        pltpu.VMEM((CH, D2), jnp.float32),
        pltpu.VMEM_SHARED((NP, D2), jnp.float32),
        pltpu.SemaphoreType.DMA,
    ],
    compiler_params=_sc_params,
)
def _deg_kernel(ei_hbm, out_hbm, dst_v, msg_v, acc_sh, sem):
    c = lax.axis_index("c")
    s = lax.axis_index("s")
    w = _worker(c, s)
    r0 = s * RPT
    _load_chunks(ei_hbm, 1, w, dst_v)

    _fill(msg_v, 0.0, D2)

    @pl.when(c != 0)
    def _():
        for i in range(RPT // CH):
            pltpu.sync_copy(msg_v, acc_sh.at[pl.ds(r0 + i * CH, CH)])

    _fill(msg_v, 1.0, D2)

    @pl.when(c == 0)
    def _():
        for i in range(RPT // CH):
            pltpu.sync_copy(msg_v, acc_sh.at[pl.ds(r0 + i * CH, CH)])

    plsc.subcore_barrier()
    nch = jnp.where(w == LASTW, LAST, FULL)

    def body(j, carry):
        @pl.when(j >= 8)
        def _():
            pltpu.make_async_copy(msg_v, acc_sh.at[dst_v.at[j - 8]], sem).wait()

        pltpu.async_copy(msg_v, acc_sh.at[dst_v.at[j]], sem, add=True)
        return carry

    lax.fori_loop(0, nch, body, 0)

    def drain(j, carry):
        pltpu.make_async_copy(msg_v, acc_sh.at[dst_v.at[j]], sem).wait()
        return carry

    lax.fori_loop(jnp.maximum(nch - 8, 0), nch, drain, 0)
    plsc.subcore_barrier()
    pltpu.sync_copy(acc_sh.at[pl.ds(r0, RPT)], out_hbm.at[pl.ds(c * NP + r0, RPT)])


def _make_seg_kernel(d, stage_in_spmem):
    @functools.partial(
        pl.kernel,
        out_type=jax.ShapeDtypeStruct((NC * NP, d), jnp.float32),
        mesh=_mesh,
        scratch_types=[
            pltpu.VMEM((FULL, CH), jnp.int32),
            pltpu.VMEM((FULL, CH), jnp.int32),
            pltpu.VMEM((2, CH, d), jnp.float32),
            pltpu.VMEM_SHARED((NP, d), jnp.float32),
            pltpu.VMEM_SHARED((NP, d) if stage_in_spmem else (8, d), jnp.float32),
            pltpu.SemaphoreType.DMA((2,)),
        ],
        compiler_params=_sc_params,
    )
    def _seg(ei_hbm, tbl_hbm, out_hbm, src_v, dst_v, msg_v, acc_sh, tbl_sh, gsem):
        c = lax.axis_index("c")
        s = lax.axis_index("s")
        w = _worker(c, s)
        r0 = s * RPT
        _load_chunks(ei_hbm, 0, w, src_v)
        _load_chunks(ei_hbm, 1, w, dst_v)
        if stage_in_spmem:
            pltpu.sync_copy(tbl_hbm.at[pl.ds(r0, RPT)], tbl_sh.at[pl.ds(r0, RPT)])
            tbl = tbl_sh
        else:
            tbl = tbl_hbm

        @pl.when(c == 0)
        def _():
            pltpu.sync_copy(tbl_hbm.at[pl.ds(r0, RPT)], acc_sh.at[pl.ds(r0, RPT)])

        @pl.when(c != 0)
        def _():
            _fill(msg_v.at[0], 0.0, d)
            for i in range(RPT // CH):
                pltpu.sync_copy(msg_v.at[0], acc_sh.at[pl.ds(r0 + i * CH, CH)])

        plsc.subcore_barrier()
        nch = jnp.where(w == LASTW, LAST, FULL)

        pltpu.async_copy(tbl.at[src_v.at[0]], msg_v.at[0], gsem.at[0])

        def body(j, carry):
            p = lax.rem(j, 2)
            q = 1 - p

            @pl.when(j + 1 < nch)
            def _():
                pltpu.async_copy(tbl.at[src_v.at[j + 1]], msg_v.at[q],
                                 gsem.at[q])

            pltpu.make_async_copy(tbl.at[src_v.at[j]], msg_v.at[p],
                                  gsem.at[p]).wait()
            pltpu.sync_copy(msg_v.at[p], acc_sh.at[dst_v.at[j]], add=True)
            return carry

        lax.fori_loop(0, nch, body, 0)
        plsc.subcore_barrier()
        pltpu.sync_copy(acc_sh.at[pl.ds(r0, RPT)], out_hbm.at[pl.ds(c * NP + r0, RPT)])

    return _seg


_seg64 = _make_seg_kernel(D_HID, stage_in_spmem=False)
_seg16 = _make_seg_kernel(D2, stage_in_spmem=True)


def _tc1_body(dis_ref, x_ref, w1_ref, h1s_ref):
    h1 = jnp.dot(x_ref[...], w1_ref[...], preferred_element_type=jnp.float32)
    h1s_ref[...] = jnp.concatenate(
        [dis_ref[:N_NODES, 0:1] * h1,
         jnp.zeros((NP - N_NODES, D_HID), jnp.float32)])


def _tc_mid_body(a1_ref, dis_ref, b1_ref, w2_ref, g_ref):
    dis = dis_ref[:, 0:1]
    h = jnp.maximum(dis * a1_ref[...] + b1_ref[...], 0.0)
    g_ref[...] = dis * jnp.dot(h, w2_ref[...], preferred_element_type=jnp.float32)


_tc1 = pl.pallas_call(
    _tc1_body,
    out_shape=jax.ShapeDtypeStruct((NP, D_HID), jnp.float32),
)

_tc_mid = pl.pallas_call(
    _tc_mid_body,
    out_shape=jax.ShapeDtypeStruct((NP, D2), jnp.float32),
)


def kernel(x, edge_index, W1, b1, W2, b2):
    ei3d = edge_index.astype(jnp.int32).reshape(2, NCHT, CH)

    w2p = jnp.zeros((D_HID, D2), jnp.float32).at[:, : W2.shape[1]].set(W2)
    b1r = b1.reshape(1, D_HID)

    degp = _deg_kernel(ei3d)
    dis16 = jnp.broadcast_to(
        lax.rsqrt(degp[:NP, 0:1] + degp[NP:, 0:1]), (NP, D2))
    h1s = _tc1(dis16, x, W1)
    s1p = _seg64(ei3d, h1s)
    a1 = s1p[:NP] + s1p[NP:]
    g = _tc_mid(a1, dis16, b1r, w2p)
    s2p = _seg16(ei3d, g)
    a2 = s2p[:N_NODES, :2] + s2p[NP:NP + N_NODES, :2]
    return dis16[:N_NODES, 0:1] * a2 + b2.reshape(1, 2)


# speedup vs baseline: 1.0622x; 1.0622x over previous
import functools

import jax
import jax.numpy as jnp
from jax import lax
from jax.experimental import pallas as pl
from jax.experimental.pallas import tpu as pltpu
from jax.experimental.pallas import tpu_sc as plsc

N_NODES = 10000
N_EDGES = 320000
D_IN = 128
D_HID = 64
D2 = 16

NC = 2
NS = 16
NW = NC * NS
CH = 128
NP = 10240
NCHT = N_EDGES // CH
FULL = 80
LASTW = NW - 1
LAST = NCHT - LASTW * FULL
RPT = NP // NS

_mesh = plsc.VectorSubcoreMesh(core_axis_name="c", subcore_axis_name="s")
_sc_params = pltpu.CompilerParams(use_tc_tiling_on_sc=False)


def _worker(c, s):
    return s * NC + c


def _load_chunks(ei_hbm, row, w, idx_v):
    @pl.when(w < LASTW)
    def _():
        pltpu.sync_copy(ei_hbm.at[row, pl.ds(w * FULL, FULL)], idx_v)

    @pl.when(w == LASTW)
    def _():
        pltpu.sync_copy(ei_hbm.at[row, pl.ds(LASTW * FULL, LAST)],
                        idx_v.at[pl.ds(0, LAST)])


def _fill(buf, value, d):
    vec = jnp.full((16,), value, jnp.float32)
    nsub = d // 16

    def body(i, carry):
        r = i // nsub
        k = (i % nsub) * 16
        buf[r, pl.ds(k, 16)] = vec
        return carry

    lax.fori_loop(0, CH * nsub, body, 0)


@functools.partial(
    pl.kernel,
    out_type=jax.ShapeDtypeStruct((NC * NP, D2), jnp.float32),
    mesh=_mesh,
    scratch_types=[
        pltpu.VMEM((FULL, CH), jnp.int32),
        pltpu.VMEM((CH, D2), jnp.float32),
        pltpu.VMEM_SHARED((NP, D2), jnp.float32),
        pltpu.SemaphoreType.DMA,
    ],
    compiler_params=_sc_params,
)
def _deg_kernel(ei_hbm, out_hbm, dst_v, msg_v, acc_sh, sem):
    c = lax.axis_index("c")
    s = lax.axis_index("s")
    w = _worker(c, s)
    r0 = s * RPT
    _load_chunks(ei_hbm, 1, w, dst_v)

    _fill(msg_v, 0.0, D2)

    @pl.when(c != 0)
    def _():
        for i in range(RPT // CH):
            pltpu.sync_copy(msg_v, acc_sh.at[pl.ds(r0 + i * CH, CH)])

    _fill(msg_v, 1.0, D2)

    @pl.when(c == 0)
    def _():
        for i in range(RPT // CH):
            pltpu.sync_copy(msg_v, acc_sh.at[pl.ds(r0 + i * CH, CH)])

    plsc.subcore_barrier()
    nch = jnp.where(w == LASTW, LAST, FULL)

    def body(j, carry):
        @pl.when(j >= 8)
        def _():
            pltpu.make_async_copy(msg_v, acc_sh.at[dst_v.at[j - 8]], sem).wait()

        pltpu.async_copy(msg_v, acc_sh.at[dst_v.at[j]], sem, add=True)
        return carry

    lax.fori_loop(0, nch, body, 0)

    def drain(j, carry):
        pltpu.make_async_copy(msg_v, acc_sh.at[dst_v.at[j]], sem).wait()
        return carry

    lax.fori_loop(jnp.maximum(nch - 8, 0), nch, drain, 0)
    plsc.subcore_barrier()
    pltpu.sync_copy(acc_sh.at[pl.ds(r0, RPT)], out_hbm.at[pl.ds(c * NP + r0, RPT)])


def _make_seg_kernel(d, stage_in_spmem):
    @functools.partial(
        pl.kernel,
        out_type=jax.ShapeDtypeStruct((NC * NP, d), jnp.float32),
        mesh=_mesh,
        scratch_types=[
            pltpu.VMEM((FULL, CH), jnp.int32),
            pltpu.VMEM((FULL, CH), jnp.int32),
            pltpu.VMEM((2, CH, d), jnp.float32),
            pltpu.VMEM_SHARED((NP, d), jnp.float32),
            pltpu.VMEM_SHARED((NP, d) if stage_in_spmem else (8, d), jnp.float32),
            pltpu.SemaphoreType.DMA((2,)),
        ],
        compiler_params=_sc_params,
    )
    def _seg(ei_hbm, tbl_hbm, out_hbm, src_v, dst_v, msg_v, acc_sh, tbl_sh, gsem):
        c = lax.axis_index("c")
        s = lax.axis_index("s")
        w = _worker(c, s)
        r0 = s * RPT
        _load_chunks(ei_hbm, 0, w, src_v)
        _load_chunks(ei_hbm, 1, w, dst_v)
        if stage_in_spmem:
            pltpu.sync_copy(tbl_hbm.at[pl.ds(r0, RPT)], tbl_sh.at[pl.ds(r0, RPT)])
            tbl = tbl_sh
        else:
            tbl = tbl_hbm

        @pl.when(c == 0)
        def _():
            pltpu.sync_copy(tbl_hbm.at[pl.ds(r0, RPT)], acc_sh.at[pl.ds(r0, RPT)])

        @pl.when(c != 0)
        def _():
            _fill(msg_v.at[0], 0.0, d)
            for i in range(RPT // CH):
                pltpu.sync_copy(msg_v.at[0], acc_sh.at[pl.ds(r0 + i * CH, CH)])

        plsc.subcore_barrier()
        nch = jnp.where(w == LASTW, LAST, FULL)

        pltpu.async_copy(tbl.at[src_v.at[0]], msg_v.at[0], gsem.at[0])

        def body(j, carry):
            p = lax.rem(j, 2)
            q = 1 - p

            @pl.when(j + 1 < nch)
            def _():
                pltpu.async_copy(tbl.at[src_v.at[j + 1]], msg_v.at[q],
                                 gsem.at[q])

            pltpu.make_async_copy(tbl.at[src_v.at[j]], msg_v.at[p],
                                  gsem.at[p]).wait()
            pltpu.sync_copy(msg_v.at[p], acc_sh.at[dst_v.at[j]], add=True)
            return carry

        lax.fori_loop(0, nch, body, 0)
        plsc.subcore_barrier()
        pltpu.sync_copy(acc_sh.at[pl.ds(r0, RPT)], out_hbm.at[pl.ds(c * NP + r0, RPT)])

    return _seg


_seg64 = _make_seg_kernel(D_HID, stage_in_spmem=False)
_seg16 = _make_seg_kernel(D2, stage_in_spmem=True)


def _tc1_body(deg_ref, x_ref, w1_ref, dis_ref, h1s_ref):
    deg = deg_ref[:NP, 0:1] + deg_ref[NP:, 0:1]
    dis = lax.rsqrt(deg)
    h1 = jnp.dot(x_ref[...], w1_ref[...], preferred_element_type=jnp.float32)
    dis_ref[...] = jnp.broadcast_to(dis, (NP, D2))
    h1s_ref[...] = jnp.concatenate(
        [dis[:N_NODES] * h1, jnp.zeros((NP - N_NODES, D_HID), jnp.float32)])


def _tc_mid_body(s1_ref, dis_ref, b1_ref, w2_ref, g_ref):
    dis = dis_ref[:, 0:1]
    a1 = s1_ref[:NP, :] + s1_ref[NP:, :]
    h = jnp.maximum(dis * a1 + b1_ref[...], 0.0)
    g_ref[...] = dis * jnp.dot(h, w2_ref[...], preferred_element_type=jnp.float32)


_tc1 = pl.pallas_call(
    _tc1_body,
    out_shape=(
        jax.ShapeDtypeStruct((NP, D2), jnp.float32),
        jax.ShapeDtypeStruct((NP, D_HID), jnp.float32),
    ),
)

_tc_mid = pl.pallas_call(
    _tc_mid_body,
    out_shape=jax.ShapeDtypeStruct((NP, D2), jnp.float32),
)


def kernel(x, edge_index, W1, b1, W2, b2):
    ei3d = edge_index.astype(jnp.int32).reshape(2, NCHT, CH)

    w2p = jnp.zeros((D_HID, D2), jnp.float32).at[:, : W2.shape[1]].set(W2)
    b1r = b1.reshape(1, D_HID)

    degp = _deg_kernel(ei3d)
    dis16, h1s = _tc1(degp, x, W1)
    s1p = _seg64(ei3d, h1s)
    g = _tc_mid(s1p, dis16, b1r, w2p)
    s2p = _seg16(ei3d, g)
    a2 = s2p[:N_NODES, :2] + s2p[NP:NP + N_NODES, :2]
    return dis16[:N_NODES, 0:1] * a2 + b2.reshape(1, 2)
